# baseline (speedup 1.0000x reference)
import functools

import numpy as np
import jax
import jax.numpy as jnp
from jax import lax
from jax.experimental import pallas as pl
from jax.experimental.pallas import tpu as pltpu
from jax.experimental.pallas import tpu_sc as plsc

_FEATURE_DIMS = (4000,) * 26
_FDIM = 4000
_F = 26
_FP = 32
_V = sum(_FEATURE_DIMS) + 1
_D = 16
_B = 4096
_SLOTS = 32
_QB = _SLOTS * _D // 128
_LIN_SLOT = _F
_BIAS_SLOT = _F + 1

_NC, _NS = 2, 16
_NW = _NC * _NS
_BPW = _B // _NW
_CB = 2
_BPB = _FP * _QB
_GRP = (_CB * _BPB) // 128
_CPG = _D // _CB

_RT = 1024
_VP = 104448
_NBLK = _VP // _RT

_mesh = plsc.VectorSubcoreMesh(core_axis_name="c", subcore_axis_name="s")


def _repack_body(tab_ref, lin_ref, bias_ref, out_ref):
    out_ref[...] = jnp.concatenate([tab_ref[t] for t in range(8)], axis=1)

    @pl.when(pl.program_id(1) == _QB - 1)
    def _():
        out_ref[:, 2 * _D:3 * _D] = jnp.pad(lin_ref[...],
                                            ((0, 0), (0, _D - 1)))
        out_ref[:, 3 * _D:4 * _D] = jnp.pad(
            jnp.full((_RT, 1), bias_ref[0, 0], jnp.float32),
            ((0, 0), (0, _D - 1)))
        out_ref[:, 4 * _D:] = jnp.zeros((_RT, 128 - 4 * _D), jnp.float32)


def _lane_sum(v):
    for sh in (8, 4, 2, 1):
        perm = lax.iota(jnp.int32, _D) ^ sh
        v = v + v.at[perm].get(mode="promise_in_bounds")
    return v


@functools.partial(
    pl.kernel,
    mesh=_mesh,
    compiler_params=pltpu.CompilerParams(use_tc_tiling_on_sc=True),
    out_type=jax.ShapeDtypeStruct((_B,), jnp.float32),
    scratch_types=[
        pltpu.VMEM((_CB * _FP,), jnp.int32),
        pltpu.VMEM((_CB * _BPB,), jnp.int32),
        pltpu.VMEM((_CB * _BPB, 128), jnp.float32),
        pltpu.VMEM((_BPW,), jnp.float32),
        pltpu.SemaphoreType.DMA,
    ],
)
def _ffm_sc(x_hbm, tab, out_hbm, xbuf, idx_v, rows_v, out_v, sem):
    cid = lax.axis_index("c")
    sid = lax.axis_index("s")
    wid = sid * _NC + cid
    b0 = wid * _BPW

    lanes = lax.iota(jnp.int32, _D)
    off_lo = _FDIM * lanes
    off_hi = jnp.where(lanes < _F - _D, _FDIM * (lanes + _D),
                       _FDIM * (lanes - (_F - _D)))

    def group(g, carry):
        def chunk(u, res):
            c = g * _CPG + u
            pltpu.sync_copy(
                x_hbm.at[pl.ds(
                    pl.multiple_of((b0 + c * _CB) * _FP, _CB * _FP),
                    _CB * _FP)],
                xbuf)
            for bl in range(_CB):
                adj_lo = xbuf[pl.ds(bl * _FP, _D)] + off_lo
                adj_hi = xbuf[pl.ds(bl * _FP + _D, _D)] + off_hi
                for q in range(_QB):
                    idx_v[pl.ds(bl * _BPB + q * _FP, _D)] = (
                        adj_lo + q * _VP)
                    idx_v[pl.ds(bl * _BPB + q * _FP + _D, _D)] = (
                        adj_hi + q * _VP)
            copies = [
                pltpu.async_copy(tab.at[idx_v.at[pl.ds(k * 128, 128)]],
                                 rows_v.at[pl.ds(k * 128, 128)], sem)
                for k in range(_GRP)
            ]
            for cp in copies:
                cp.wait()

            for bl in range(_CB):
                gb = bl * _BPB

                zero = jnp.zeros((_D,), jnp.float32)

                def outer(i, carry):
                    a0, a1, a2, a3, dg = carry
                    blk_a = gb + (i >> 3) * _FP
                    sub_a = pl.multiple_of((i & 7) * _D, _D)
                    accs = [a0, a1, a2, a3]
                    for jq in range(_QB):
                        blk_b = gb + jq * _FP + i
                        for j8 in range(8):
                            j = 8 * jq + j8
                            if j >= _F:
                                break
                            a = rows_v[blk_a + j, pl.ds(sub_a, _D)]
                            b = rows_v[blk_b, pl.ds(j8 * _D, _D)]
                            accs[j % 4] = accs[j % 4] + a * b
                    dv = rows_v[blk_a + i, pl.ds(sub_a, _D)]
                    return (accs[0], accs[1], accs[2], accs[3],
                            dg + dv * dv)

                a0, a1, a2, a3, dg = lax.fori_loop(
                    0, _F, outer, (zero, zero, zero, zero, zero))
                acc = ((a0 + a1) + (a2 + a3) - dg) * 0.5

                def lin(f, acc):
                    return acc + rows_v[gb + 3 * _FP + f,
                                        pl.ds((_LIN_SLOT % 8) * _D, _D)]

                acc = lax.fori_loop(0, _F, lin, acc)
                acc = acc + rows_v[gb + 3 * _FP,
                                   pl.ds((_BIAS_SLOT % 8) * _D, _D)]
                zvec = _lane_sum(acc)
                res = jnp.where(lanes == u * _CB + bl, zvec, res)
            return res

        res = lax.fori_loop(0, _CPG, chunk, jnp.zeros((_D,), jnp.float32))
        out_v[pl.ds(pl.multiple_of(g * _D, _D), _D)] = (
            1.0 / (1.0 + jnp.exp(-res)))
        return carry

    lax.fori_loop(0, _BPW // _D, group, 0)
    pltpu.sync_copy(out_v, out_hbm.at[pl.ds(b0, _BPW)])


def kernel(x, tables, lin_w, lin_b):
    xi = x.astype(jnp.int32)
    x32 = jnp.concatenate([xi, xi[:, :_FP - _F]], axis=1)
    t2 = pl.pallas_call(
        _repack_body,
        grid=(_NBLK, _QB),
        in_specs=[
            pl.BlockSpec((8, _RT, _D), lambda i, q: (q, i, 0)),
            pl.BlockSpec((_RT, 1), lambda i, q: (i, 0)),
            pl.BlockSpec((1, 1), lambda i, q: (0, 0)),
        ],
        out_specs=pl.BlockSpec((_RT, 128), lambda i, q: (q * _NBLK + i, 0)),
        out_shape=jax.ShapeDtypeStruct((_QB * _VP, 128), jnp.float32),
    )(tables, lin_w, lin_b.reshape(1, 1))
    return _ffm_sc(x32.reshape(-1), t2)


# speedup vs baseline: 1.1640x; 1.1640x over previous
import functools

import numpy as np
import jax
import jax.numpy as jnp
from jax import lax
from jax.experimental import pallas as pl
from jax.experimental.pallas import tpu as pltpu
from jax.experimental.pallas import tpu_sc as plsc

_FEATURE_DIMS = (4000,) * 26
_FDIM = 4000
_F = 26
_FP = 32
_V = sum(_FEATURE_DIMS) + 1
_D = 16
_B = 4096
_SLOTS = 32
_QB = _SLOTS * _D // 128
_LIN_SLOT = _F
_BIAS_SLOT = _F + 1

_NC, _NS = 2, 16
_NW = _NC * _NS
_BPW = _B // _NW
_CB = 2
_BPB = _FP * _QB
_GRP = (_CB * _BPB) // 128
_CPG = _D // _CB

_mesh = plsc.VectorSubcoreMesh(core_axis_name="c", subcore_axis_name="s")


def _lane_sum(v):
    for sh in (8, 4, 2, 1):
        perm = lax.iota(jnp.int32, _D) ^ sh
        v = v + v.at[perm].get(mode="promise_in_bounds")
    return v


@functools.partial(
    pl.kernel,
    mesh=_mesh,
    compiler_params=pltpu.CompilerParams(use_tc_tiling_on_sc=True),
    out_type=jax.ShapeDtypeStruct((_B,), jnp.float32),
    scratch_types=[
        pltpu.VMEM((_CB * _FP,), jnp.int32),
        pltpu.VMEM((_CB * _BPB,), jnp.int32),
        pltpu.VMEM((_CB * _BPB, 128), jnp.float32),
        pltpu.VMEM((_BPW,), jnp.float32),
        pltpu.SemaphoreType.DMA,
    ],
)
def _ffm_sc(x_hbm, tab, out_hbm, xbuf, idx_v, rows_v, out_v, sem):
    cid = lax.axis_index("c")
    sid = lax.axis_index("s")
    wid = sid * _NC + cid
    b0 = wid * _BPW

    lanes = lax.iota(jnp.int32, _D)
    off_lo = _FDIM * lanes
    off_hi = jnp.where(lanes < _F - _D, _FDIM * (lanes + _D),
                       _FDIM * (lanes - (_F - _D)))

    def group(g, carry):
        def chunk(u, res):
            c = g * _CPG + u
            pltpu.sync_copy(
                x_hbm.at[pl.ds(
                    pl.multiple_of((b0 + c * _CB) * _FP, _CB * _FP),
                    _CB * _FP)],
                xbuf)
            for bl in range(_CB):
                adj_lo = (xbuf[pl.ds(bl * _FP, _D)] + off_lo) * _QB
                adj_hi = (xbuf[pl.ds(bl * _FP + _D, _D)] + off_hi) * _QB
                for q in range(_QB):
                    idx_v[pl.ds(bl * _BPB + q * _FP, _D)] = adj_lo + q
                    idx_v[pl.ds(bl * _BPB + q * _FP + _D, _D)] = adj_hi + q
            copies = [
                pltpu.async_copy(tab.at[idx_v.at[pl.ds(k * 128, 128)]],
                                 rows_v.at[pl.ds(k * 128, 128)], sem)
                for k in range(_GRP)
            ]
            for cp in copies:
                cp.wait()

            for bl in range(_CB):
                gb = bl * _BPB

                zero = jnp.zeros((_D,), jnp.float32)

                def outer(i, carry):
                    a0, a1, a2, a3, dg = carry
                    blk_a = gb + (i >> 3) * _FP
                    sub_a = pl.multiple_of((i & 7) * _D, _D)
                    accs = [a0, a1, a2, a3]
                    for jq in range(_QB):
                        blk_b = gb + jq * _FP + i
                        for j8 in range(8):
                            j = 8 * jq + j8
                            if j >= _F:
                                break
                            a = rows_v[blk_a + j, pl.ds(sub_a, _D)]
                            b = rows_v[blk_b, pl.ds(j8 * _D, _D)]
                            accs[j % 4] = accs[j % 4] + a * b
                    dv = rows_v[blk_a + i, pl.ds(sub_a, _D)]
                    return (accs[0], accs[1], accs[2], accs[3],
                            dg + dv * dv)

                a0, a1, a2, a3, dg = lax.fori_loop(
                    0, _F, outer, (zero, zero, zero, zero, zero))
                acc = ((a0 + a1) + (a2 + a3) - dg) * 0.5

                def lin(f, acc):
                    return acc + rows_v[gb + 3 * _FP + f,
                                        pl.ds((_LIN_SLOT % 8) * _D, _D)]

                acc = lax.fori_loop(0, _F, lin, acc)
                acc = acc + rows_v[gb + 3 * _FP,
                                   pl.ds((_BIAS_SLOT % 8) * _D, _D)]
                zvec = _lane_sum(acc)
                res = jnp.where(lanes == u * _CB + bl, zvec, res)
            return res

        res = lax.fori_loop(0, _CPG, chunk, jnp.zeros((_D,), jnp.float32))
        out_v[pl.ds(pl.multiple_of(g * _D, _D), _D)] = (
            1.0 / (1.0 + jnp.exp(-res)))
        return carry

    lax.fori_loop(0, _BPW // _D, group, 0)
    pltpu.sync_copy(out_v, out_hbm.at[pl.ds(b0, _BPW)])


def kernel(x, tables, lin_w, lin_b):
    xi = x.astype(jnp.int32)
    x32 = jnp.concatenate([xi, xi[:, :_FP - _F]], axis=1)
    tp = jnp.transpose(tables, (1, 0, 2)).reshape(_V, _F * _D)
    lin_col = jnp.pad(lin_w, ((0, 0), (0, _D - 1)))
    bias_col = jnp.pad(
        jnp.broadcast_to(lin_b.reshape(1, 1), (_V, 1)),
        ((0, 0), (0, _D - 1)))
    zpad = jnp.zeros((_V, (_SLOTS - _F - 2) * _D), jnp.float32)
    t2 = jnp.concatenate([tp, lin_col, bias_col, zpad], axis=1)
    t2 = t2.reshape(_V * _QB, 128)
    return _ffm_sc(x32.reshape(-1), t2)
